# baseline (speedup 1.0000x reference)
import functools

import jax
import jax.numpy as jnp
from jax import lax
from jax.experimental import pallas as pl
from jax.experimental.pallas import tpu as pltpu
from jax.experimental.pallas import tpu_sc as plsc

_B = 4
_NB = 2048
_F = 128
_M = 512
_R2 = 0.2 * 0.2
_K = 64
_H1 = 128
_OUT = 128
_NEG = float("-inf")

_MB = 128
_MBD = 64
_RB = 512


def _fps_body(posS_ref, psx_ref, psy_ref, psz_ref):
    ps = posS_ref[...]
    posx = ps[0:_B, :]
    posy = ps[_B:2 * _B, :]
    posz = ps[2 * _B:3 * _B, :]
    iota = lax.broadcasted_iota(jnp.int32, (_B, _NB), 1)
    iota12 = lax.broadcasted_iota(jnp.int32, (3 * _B, _NB), 1)

    def step(i, carry):
        dists, cur = carry
        cur12 = jnp.concatenate([cur, cur, cur])
        oh12 = iota12 == cur12[:, None]
        s = jnp.sum(jnp.where(oh12, ps, 0.0), axis=1)
        px = s[0:_B]
        py = s[_B:2 * _B]
        pz = s[2 * _B:3 * _B]
        psx_ref[pl.ds(i, 1), :] = px[None, :]
        psy_ref[pl.ds(i, 1), :] = py[None, :]
        psz_ref[pl.ds(i, 1), :] = pz[None, :]
        d = (posx - px[:, None]) ** 2 + (posy - py[:, None]) ** 2 \
            + (posz - pz[:, None]) ** 2
        dists = jnp.minimum(dists, d)
        mx = jnp.max(dists, axis=1)
        nxt = jnp.min(jnp.where(dists == mx[:, None], iota, _NB), axis=1)
        return dists, nxt.astype(jnp.int32)

    lax.fori_loop(
        0, _M, step,
        (jnp.full((_B, _NB), jnp.inf, jnp.float32),
         jnp.zeros((_B,), jnp.int32)))


def _fps_call(posS):
    return pl.pallas_call(
        _fps_body,
        out_shape=[
            jax.ShapeDtypeStruct((_M, _B), jnp.float32),
            jax.ShapeDtypeStruct((_M, _B), jnp.float32),
            jax.ShapeDtypeStruct((_M, _B), jnp.float32),
        ],
    )(posS)


def _q_body(x_ref, pp_ref, w1x_ref, w1p_ref, b1_ref, q_ref):
    q_ref[...] = (
        jnp.dot(x_ref[...], w1x_ref[...], preferred_element_type=jnp.float32)
        + jnp.dot(pp_ref[...], w1p_ref[...], preferred_element_type=jnp.float32)
        + b1_ref[...])


def _q_call(x, pp, w1x, w1p, b1row):
    n = x.shape[0]
    return pl.pallas_call(
        _q_body,
        grid=(n // _RB,),
        in_specs=[
            pl.BlockSpec((_RB, _F), lambda i: (i, 0)),
            pl.BlockSpec((_RB, 8), lambda i: (i, 0)),
            pl.BlockSpec((_F, _H1), lambda i: (0, 0)),
            pl.BlockSpec((8, _H1), lambda i: (0, 0)),
            pl.BlockSpec((1, _H1), lambda i: (0, 0)),
        ],
        out_specs=pl.BlockSpec((_RB, _H1), lambda i: (i, 0)),
        out_shape=jax.ShapeDtypeStruct((n, _H1), jnp.float32),
    )(x, pp, w1x, w1p, b1row)


def _topk_body(posT_ref, psx_ref, psy_ref, psz_ref, nidx_ref, nval_ref,
               sel_ref):
    p = posT_ref[...]
    posx = p[0, 0, :]
    posy = p[0, 1, :]
    posz = p[0, 2, :]
    px = psx_ref[...].reshape(_MB)
    py = psy_ref[...].reshape(_MB)
    pz = psz_ref[...].reshape(_MB)
    d2 = (px[:, None] - posx[None, :]) ** 2 \
        + (py[:, None] - posy[None, :]) ** 2 \
        + (pz[:, None] - posz[None, :]) ** 2
    sel_ref[...] = jnp.where(d2 <= _R2, -d2, _NEG)
    iota = lax.broadcasted_iota(jnp.int32, (_MB, _NB), 1)

    def body(k, _):
        sel = sel_ref[...]
        mx = jnp.max(sel, axis=1)
        idxk = jnp.min(jnp.where(sel == mx[:, None], iota, _NB), axis=1)
        nidx_ref[:, :, pl.ds(k, 1), :] = \
            idxk[None, None, None, :].astype(jnp.int32)
        nval_ref[:, :, pl.ds(k, 1), :] = mx[None, None, None, :]
        sel_ref[...] = jnp.where(iota == idxk[:, None], _NEG, sel)
        return 0

    lax.fori_loop(0, _K, body, 0)


def _topk_call(posT, psx4, psy4, psz4):
    nmb = _M // _MB
    return pl.pallas_call(
        _topk_body,
        grid=(_B, nmb),
        in_specs=[
            pl.BlockSpec((1, 3, _NB), lambda b, m: (b, 0, 0)),
            pl.BlockSpec((1, 1, 1, _MB), lambda b, m: (b, m, 0, 0)),
            pl.BlockSpec((1, 1, 1, _MB), lambda b, m: (b, m, 0, 0)),
            pl.BlockSpec((1, 1, 1, _MB), lambda b, m: (b, m, 0, 0)),
        ],
        out_specs=[
            pl.BlockSpec((1, 1, _K, _MB), lambda b, m: (b, m, 0, 0)),
            pl.BlockSpec((1, 1, _K, _MB), lambda b, m: (b, m, 0, 0)),
        ],
        out_shape=[
            jax.ShapeDtypeStruct((_B, nmb, _K, _MB), jnp.int32),
            jax.ShapeDtypeStruct((_B, nmb, _K, _MB), jnp.float32),
        ],
        scratch_shapes=[pltpu.VMEM((_MB, _NB), jnp.float32)],
    )(posT, psx4, psy4, psz4)


def _gather_call(q, idx3):
    num_cores, num_subcores = 2, 16
    nw = num_cores * num_subcores
    n = idx3.shape[0] * idx3.shape[1] * idx3.shape[2]
    per_w = n // nw
    ch = idx3.shape[2]
    nch = idx3.shape[1]
    mesh = plsc.VectorSubcoreMesh(core_axis_name="c", subcore_axis_name="s",
                                  num_cores=num_cores,
                                  num_subcores=num_subcores)

    nbuf = 4

    @functools.partial(
        pl.kernel,
        mesh=mesh,
        out_type=jax.ShapeDtypeStruct((n, _H1), jnp.float32),
        scratch_types=(
            [pltpu.VMEM((nch, ch), jnp.int32)]
            + [pltpu.VMEM((ch, _H1), jnp.float32) for _ in range(nbuf)]
            + [pltpu.SemaphoreType.DMA for _ in range(2 * nbuf)]
        ),
    )
    def gk(q_hbm, idx_hbm, out_hbm, idx_v, *bufs_and_sems):
        rows = bufs_and_sems[:nbuf]
        gsem = bufs_and_sems[nbuf:2 * nbuf]
        wsem = bufs_and_sems[2 * nbuf:]
        wid = lax.axis_index("s") * num_cores + lax.axis_index("c")
        base = wid * per_w
        pltpu.sync_copy(idx_hbm.at[wid], idx_v)

        def outer(t, _):
            for b in range(nbuf):
                j = nbuf * t + b

                @pl.when(t > 0)
                def _():
                    pltpu.make_async_copy(
                        rows[b], out_hbm.at[pl.ds(base, ch)], wsem[b]).wait()

                pltpu.async_copy(q_hbm.at[idx_v.at[j]], rows[b], gsem[b])
            for b in range(nbuf):
                j = nbuf * t + b
                pltpu.make_async_copy(
                    q_hbm.at[pl.ds(0, ch)], rows[b], gsem[b]).wait()
                pltpu.async_copy(
                    rows[b], out_hbm.at[pl.ds(base + j * ch, ch)], wsem[b])
            return 0

        lax.fori_loop(0, nch // nbuf, outer, 0)
        for b in range(nbuf):
            pltpu.make_async_copy(
                rows[b], out_hbm.at[pl.ds(base, ch)], wsem[b]).wait()

    return gk(q, idx3)


def _mlp_body(g_ref, nval_ref, psx_ref, psy_ref, psz_ref, w1p_ref, w2_ref,
              b2_ref, out_ref):
    px = psx_ref[...].reshape(_MBD)
    py = psy_ref[...].reshape(_MBD)
    pz = psz_ref[...].reshape(_MBD)
    w1p = w1p_ref[...]
    c = (px[:, None] * w1p[0][None, :]
         + py[:, None] * w1p[1][None, :]
         + pz[:, None] * w1p[2][None, :])
    g = g_ref[...].reshape(_MBD, _K, _H1)
    c3 = lax.broadcast_in_dim(c, (_MBD, _K, _H1), (0, 2))
    h1 = jnp.maximum(g - c3, 0.0)
    h = jnp.dot(h1.reshape(_MBD * _K, _H1), w2_ref[...],
                preferred_element_type=jnp.float32)
    pen = jnp.where(nval_ref[...].reshape(_MBD, _K) > _NEG, 0.0, _NEG)
    h = h.reshape(_MBD, _K, _OUT) \
        + lax.broadcast_in_dim(pen, (_MBD, _K, _OUT), (0, 1))
    out_ref[...] = (jnp.max(h, axis=1) + b2_ref[...])[None]


def _mlp_call(g4, nval4, psxd, psyd, pszd, w1p, w2, b2row):
    nblk = (_B * _M) // _MBD
    return pl.pallas_call(
        _mlp_body,
        grid=(nblk,),
        in_specs=[
            pl.BlockSpec((1, _MBD * _K, _H1), lambda i: (i, 0, 0)),
            pl.BlockSpec((1, _MBD, _K), lambda i: (i, 0, 0)),
            pl.BlockSpec((1, 1, _MBD), lambda i: (i, 0, 0)),
            pl.BlockSpec((1, 1, _MBD), lambda i: (i, 0, 0)),
            pl.BlockSpec((1, 1, _MBD), lambda i: (i, 0, 0)),
            pl.BlockSpec((8, _H1), lambda i: (0, 0)),
            pl.BlockSpec((_H1, _OUT), lambda i: (0, 0)),
            pl.BlockSpec((1, _OUT), lambda i: (0, 0)),
        ],
        out_specs=pl.BlockSpec((1, _MBD, _OUT), lambda i: (i, 0, 0)),
        out_shape=jax.ShapeDtypeStruct((nblk, _MBD, _OUT), jnp.float32),
    )(g4, nval4, psxd, psyd, pszd, w1p, w2, b2row)


def kernel(x, pos, batch, W1, b1, W2, b2):
    posT = pos.reshape(_B, _NB, 3).transpose(0, 2, 1)
    posS = posT.transpose(1, 0, 2).reshape(3 * _B, _NB)
    psx, psy, psz = _fps_call(posS)

    pp = jnp.pad(pos, ((0, 0), (0, 5)))
    w1x = W1[:_F]
    w1p = jnp.pad(W1[_F:], ((0, 5), (0, 0)))
    q = _q_call(x, pp, w1x, w1p, b1.reshape(1, _H1))

    nmb = _M // _MB
    psx4 = psx.T.reshape(_B, nmb, 1, _MB)
    psy4 = psy.T.reshape(_B, nmb, 1, _MB)
    psz4 = psz.T.reshape(_B, nmb, 1, _MB)
    nidx, nval = _topk_call(posT, psx4, psy4, psz4)

    nbr = nidx.transpose(0, 1, 3, 2).reshape(_B, _M, _K)
    gidx = nbr + (jnp.arange(_B, dtype=jnp.int32) * _NB)[:, None, None]
    idx3 = gidx.reshape(32, (_B * _M * _K) // (32 * 128), 128)
    g = _gather_call(q, idx3)

    nblk = (_B * _M) // _MBD
    g4 = g.reshape(nblk, _MBD * _K, _H1)
    nval4 = nval.transpose(0, 1, 3, 2).reshape(nblk, _MBD, _K)
    psxd = psx.T.reshape(nblk, 1, _MBD)
    psyd = psy.T.reshape(nblk, 1, _MBD)
    pszd = psz.T.reshape(nblk, 1, _MBD)
    x_out = _mlp_call(g4, nval4, psxd, psyd, pszd, w1p, W2,
                      b2.reshape(1, _OUT)).reshape(_B * _M, _OUT)

    pos_out = jnp.stack([psx, psy, psz], axis=-1)
    pos_out = pos_out.transpose(1, 0, 2).reshape(_B * _M, 3)
    batch_out = jnp.repeat(jnp.arange(_B, dtype=batch.dtype), _M)
    return x_out, pos_out, batch_out


# speedup vs baseline: 1.0912x; 1.0912x over previous
import functools

import jax
import jax.numpy as jnp
from jax import lax
from jax.experimental import pallas as pl
from jax.experimental.pallas import tpu as pltpu
from jax.experimental.pallas import tpu_sc as plsc

_B = 4
_NB = 2048
_F = 128
_M = 512
_R2 = 0.2 * 0.2
_K = 64
_H1 = 128
_OUT = 128
_NEG = float("-inf")

_MB = 128
_MBD = 64
_RB = 512


def _fps_body(posT_ref, idx_ref, psx_ref, psy_ref, psz_ref):
    p = posT_ref[...]
    posx = p[:, 0, :]
    posy = p[:, 1, :]
    posz = p[:, 2, :]
    iota = lax.broadcasted_iota(jnp.int32, (_B, _NB), 1)

    def step(i, carry):
        dists, cur = carry
        oh = iota == cur[:, None]
        px = jnp.sum(jnp.where(oh, posx, 0.0), axis=1)
        py = jnp.sum(jnp.where(oh, posy, 0.0), axis=1)
        pz = jnp.sum(jnp.where(oh, posz, 0.0), axis=1)
        idx_ref[pl.ds(i, 1), :] = cur[None, :]
        psx_ref[pl.ds(i, 1), :] = px[None, :]
        psy_ref[pl.ds(i, 1), :] = py[None, :]
        psz_ref[pl.ds(i, 1), :] = pz[None, :]
        d = (posx - px[:, None]) ** 2 + (posy - py[:, None]) ** 2 \
            + (posz - pz[:, None]) ** 2
        dists = jnp.minimum(dists, d)
        nxt = jnp.argmax(dists, axis=1)
        return dists, nxt.astype(jnp.int32)

    lax.fori_loop(
        0, _M, step,
        (jnp.full((_B, _NB), jnp.inf, jnp.float32),
         jnp.zeros((_B,), jnp.int32)))


def _fps_call(posT):
    return pl.pallas_call(
        _fps_body,
        out_shape=[
            jax.ShapeDtypeStruct((_M, _B), jnp.int32),
            jax.ShapeDtypeStruct((_M, _B), jnp.float32),
            jax.ShapeDtypeStruct((_M, _B), jnp.float32),
            jax.ShapeDtypeStruct((_M, _B), jnp.float32),
        ],
    )(posT)


def _q_body(x_ref, pp_ref, w1x_ref, w1p_ref, b1_ref, q_ref):
    q_ref[...] = (
        jnp.dot(x_ref[...], w1x_ref[...], preferred_element_type=jnp.float32)
        + jnp.dot(pp_ref[...], w1p_ref[...], preferred_element_type=jnp.float32)
        + b1_ref[...])


def _q_call(x, pp, w1x, w1p, b1row):
    n = x.shape[0]
    return pl.pallas_call(
        _q_body,
        grid=(n // _RB,),
        in_specs=[
            pl.BlockSpec((_RB, _F), lambda i: (i, 0)),
            pl.BlockSpec((_RB, 8), lambda i: (i, 0)),
            pl.BlockSpec((_F, _H1), lambda i: (0, 0)),
            pl.BlockSpec((8, _H1), lambda i: (0, 0)),
            pl.BlockSpec((1, _H1), lambda i: (0, 0)),
        ],
        out_specs=pl.BlockSpec((_RB, _H1), lambda i: (i, 0)),
        out_shape=jax.ShapeDtypeStruct((n, _H1), jnp.float32),
    )(x, pp, w1x, w1p, b1row)


def _topk_body(posT_ref, psx_ref, psy_ref, psz_ref, nidx_ref, nval_ref,
               sel_ref):
    p = posT_ref[...]
    posx = p[0, 0, :]
    posy = p[0, 1, :]
    posz = p[0, 2, :]
    px = psx_ref[...].reshape(_MB)
    py = psy_ref[...].reshape(_MB)
    pz = psz_ref[...].reshape(_MB)
    d2 = (px[:, None] - posx[None, :]) ** 2 \
        + (py[:, None] - posy[None, :]) ** 2 \
        + (pz[:, None] - posz[None, :]) ** 2
    sel_ref[...] = jnp.where(d2 <= _R2, -d2, _NEG)
    iota = lax.broadcasted_iota(jnp.int32, (_MB, _NB), 1)

    def body(k, _):
        sel = sel_ref[...]
        mx = jnp.max(sel, axis=1)
        idxk = jnp.min(jnp.where(sel == mx[:, None], iota, _NB), axis=1)
        nidx_ref[:, :, pl.ds(k, 1), :] = \
            idxk[None, None, None, :].astype(jnp.int32)
        nval_ref[:, :, pl.ds(k, 1), :] = mx[None, None, None, :]
        sel_ref[...] = jnp.where(iota == idxk[:, None], _NEG, sel)
        return 0

    lax.fori_loop(0, _K, body, 0)


def _topk_call(posT, psx4, psy4, psz4):
    nmb = _M // _MB
    return pl.pallas_call(
        _topk_body,
        grid=(_B, nmb),
        in_specs=[
            pl.BlockSpec((1, 3, _NB), lambda b, m: (b, 0, 0)),
            pl.BlockSpec((1, 1, 1, _MB), lambda b, m: (b, m, 0, 0)),
            pl.BlockSpec((1, 1, 1, _MB), lambda b, m: (b, m, 0, 0)),
            pl.BlockSpec((1, 1, 1, _MB), lambda b, m: (b, m, 0, 0)),
        ],
        out_specs=[
            pl.BlockSpec((1, 1, _K, _MB), lambda b, m: (b, m, 0, 0)),
            pl.BlockSpec((1, 1, _K, _MB), lambda b, m: (b, m, 0, 0)),
        ],
        out_shape=[
            jax.ShapeDtypeStruct((_B, nmb, _K, _MB), jnp.int32),
            jax.ShapeDtypeStruct((_B, nmb, _K, _MB), jnp.float32),
        ],
        scratch_shapes=[pltpu.VMEM((_MB, _NB), jnp.float32)],
    )(posT, psx4, psy4, psz4)


def _gather_call(q, idx3):
    num_cores, num_subcores = 2, 16
    nw = num_cores * num_subcores
    n = idx3.shape[0] * idx3.shape[1] * idx3.shape[2]
    per_w = n // nw
    ch = idx3.shape[2]
    nch = idx3.shape[1]
    mesh = plsc.VectorSubcoreMesh(core_axis_name="c", subcore_axis_name="s",
                                  num_cores=num_cores,
                                  num_subcores=num_subcores)

    nbuf = 4

    @functools.partial(
        pl.kernel,
        mesh=mesh,
        out_type=jax.ShapeDtypeStruct((n, _H1), jnp.float32),
        scratch_types=(
            [pltpu.VMEM((nch, ch), jnp.int32)]
            + [pltpu.VMEM((ch, _H1), jnp.float32) for _ in range(nbuf)]
            + [pltpu.SemaphoreType.DMA for _ in range(2 * nbuf)]
        ),
    )
    def gk(q_hbm, idx_hbm, out_hbm, idx_v, *bufs_and_sems):
        rows = bufs_and_sems[:nbuf]
        gsem = bufs_and_sems[nbuf:2 * nbuf]
        wsem = bufs_and_sems[2 * nbuf:]
        wid = lax.axis_index("s") * num_cores + lax.axis_index("c")
        base = wid * per_w
        pltpu.sync_copy(idx_hbm.at[wid], idx_v)

        def outer(t, _):
            for b in range(nbuf):
                j = nbuf * t + b

                @pl.when(t > 0)
                def _():
                    pltpu.make_async_copy(
                        rows[b], out_hbm.at[pl.ds(base, ch)], wsem[b]).wait()

                pltpu.async_copy(q_hbm.at[idx_v.at[j]], rows[b], gsem[b])
            for b in range(nbuf):
                j = nbuf * t + b
                pltpu.make_async_copy(
                    q_hbm.at[pl.ds(0, ch)], rows[b], gsem[b]).wait()
                pltpu.async_copy(
                    rows[b], out_hbm.at[pl.ds(base + j * ch, ch)], wsem[b])
            return 0

        lax.fori_loop(0, nch // nbuf, outer, 0)
        for b in range(nbuf):
            pltpu.make_async_copy(
                rows[b], out_hbm.at[pl.ds(base, ch)], wsem[b]).wait()

    return gk(q, idx3)


def _mlp_body(g_ref, nval_ref, psx_ref, psy_ref, psz_ref, w1p_ref, w2_ref,
              b2_ref, out_ref):
    px = psx_ref[...].reshape(_MBD)
    py = psy_ref[...].reshape(_MBD)
    pz = psz_ref[...].reshape(_MBD)
    w1p = w1p_ref[...]
    c = (px[:, None] * w1p[0][None, :]
         + py[:, None] * w1p[1][None, :]
         + pz[:, None] * w1p[2][None, :])
    g = g_ref[...].reshape(_MBD, _K, _H1)
    c3 = lax.broadcast_in_dim(c, (_MBD, _K, _H1), (0, 2))
    h1 = jnp.maximum(g - c3, 0.0)
    h = jnp.dot(h1.reshape(_MBD * _K, _H1), w2_ref[...],
                preferred_element_type=jnp.float32)
    pen = jnp.where(nval_ref[...].reshape(_MBD, _K) > _NEG, 0.0, _NEG)
    h = h.reshape(_MBD, _K, _OUT) \
        + lax.broadcast_in_dim(pen, (_MBD, _K, _OUT), (0, 1))
    out_ref[...] = (jnp.max(h, axis=1) + b2_ref[...])[None]


def _mlp_call(g4, nval4, psxd, psyd, pszd, w1p, w2, b2row):
    nblk = (_B * _M) // _MBD
    return pl.pallas_call(
        _mlp_body,
        grid=(nblk,),
        in_specs=[
            pl.BlockSpec((1, _MBD * _K, _H1), lambda i: (i, 0, 0)),
            pl.BlockSpec((1, _MBD, _K), lambda i: (i, 0, 0)),
            pl.BlockSpec((1, 1, _MBD), lambda i: (i, 0, 0)),
            pl.BlockSpec((1, 1, _MBD), lambda i: (i, 0, 0)),
            pl.BlockSpec((1, 1, _MBD), lambda i: (i, 0, 0)),
            pl.BlockSpec((8, _H1), lambda i: (0, 0)),
            pl.BlockSpec((_H1, _OUT), lambda i: (0, 0)),
            pl.BlockSpec((1, _OUT), lambda i: (0, 0)),
        ],
        out_specs=pl.BlockSpec((1, _MBD, _OUT), lambda i: (i, 0, 0)),
        out_shape=jax.ShapeDtypeStruct((nblk, _MBD, _OUT), jnp.float32),
    )(g4, nval4, psxd, psyd, pszd, w1p, w2, b2row)


def kernel(x, pos, batch, W1, b1, W2, b2):
    posT = pos.reshape(_B, _NB, 3).transpose(0, 2, 1)
    idx_sb, psx, psy, psz = _fps_call(posT)

    pp = jnp.pad(pos, ((0, 0), (0, 5)))
    w1x = W1[:_F]
    w1p = jnp.pad(W1[_F:], ((0, 5), (0, 0)))
    q = _q_call(x, pp, w1x, w1p, b1.reshape(1, _H1))

    nmb = _M // _MB
    psx4 = psx.T.reshape(_B, nmb, 1, _MB)
    psy4 = psy.T.reshape(_B, nmb, 1, _MB)
    psz4 = psz.T.reshape(_B, nmb, 1, _MB)
    nidx, nval = _topk_call(posT, psx4, psy4, psz4)

    nbr = nidx.transpose(0, 1, 3, 2).reshape(_B, _M, _K)
    gidx = nbr + (jnp.arange(_B, dtype=jnp.int32) * _NB)[:, None, None]
    idx3 = gidx.reshape(32, (_B * _M * _K) // (32 * 128), 128)
    g = _gather_call(q, idx3)

    nblk = (_B * _M) // _MBD
    g4 = g.reshape(nblk, _MBD * _K, _H1)
    nval4 = nval.transpose(0, 1, 3, 2).reshape(nblk, _MBD, _K)
    psxd = psx.T.reshape(nblk, 1, _MBD)
    psyd = psy.T.reshape(nblk, 1, _MBD)
    pszd = psz.T.reshape(nblk, 1, _MBD)
    x_out = _mlp_call(g4, nval4, psxd, psyd, pszd, w1p, W2,
                      b2.reshape(1, _OUT)).reshape(_B * _M, _OUT)

    pos_out = jnp.stack([psx, psy, psz], axis=-1)
    pos_out = pos_out.transpose(1, 0, 2).reshape(_B * _M, 3)
    batch_out = jnp.repeat(jnp.arange(_B, dtype=batch.dtype), _M)
    return x_out, pos_out, batch_out
